# baseline (speedup 1.0000x reference)
import functools

import jax
import jax.numpy as jnp
from jax import lax
from jax.experimental import pallas as pl
from jax.experimental.pallas import tpu as pltpu
from jax.experimental.pallas import tpu_sc as plsc

NC, NS = 2, 16
NW = NC * NS
TB = 128
PITCH = TB + 1
DEPTH = 4


@functools.lru_cache(maxsize=None)
def _build(hist, batch, vocab, d_model):
    nb = batch // TB
    nblk = hist * nb
    per_w = nblk // NW
    assert nblk % NW == 0 and per_w % DEPTH == 0 and per_w >= 2 * DEPTH

    mesh = plsc.VectorSubcoreMesh(core_axis_name="c", subcore_axis_name="s")

    @functools.partial(
        pl.kernel,
        out_type=jax.ShapeDtypeStruct((hist, d_model, batch), jnp.float32),
        mesh=mesh,
        scratch_types=[
            pltpu.VMEM((DEPTH, TB), jnp.int32),
            pltpu.VMEM((DEPTH, TB), jnp.int32),
            pltpu.VMEM((DEPTH, TB), jnp.int32),
            pltpu.VMEM((DEPTH, TB, 2 * d_model), jnp.float32),
            pltpu.VMEM((2, d_model, PITCH), jnp.float32),
        ] + [pltpu.SemaphoreType.DMA] * (2 * DEPTH + 2),
        compiler_params=pltpu.CompilerParams(
            use_tc_tiling_on_sc=True, needs_layout_passes=False,
            disable_bounds_checks=True),
    )
    def embed(tab2_hbm, xt_hbm, out_hbm, idx_v, off_v, p_v, rows_v, tr_v,
              *sems):
        sem_i = sems[:DEPTH]
        sem_g = sems[DEPTH:2 * DEPTH]
        sem_o = sems[2 * DEPTH:]
        wid = lax.axis_index("s") * NC + lax.axis_index("c")
        iota = lax.iota(jnp.int32, 16)
        rvec = [16 * k + iota for k in range(d_model // 16)]

        def hc(g):
            blk = wid + g * NW
            return blk // nb, blk % nb

        def idx_cp(g, s):
            h, c = hc(g)
            return pltpu.make_async_copy(
                xt_hbm.at[h, pl.ds(c * TB, TB)], idx_v.at[s], sem_i[s])

        def gat_cp(s):
            return pltpu.make_async_copy(
                tab2_hbm.at[p_v.at[s]], rows_v.at[s], sem_g[s])

        def out_cp(g, s2):
            h, c = hc(g)
            return pltpu.make_async_copy(
                tr_v.at[s2, :, pl.ds(0, TB)],
                out_hbm.at[h, :, pl.ds(c * TB, TB)], sem_o[s2])

        def compute_p(s):
            for l in range(TB // 16):
                v = idx_v[s, pl.ds(16 * l, 16)]
                p_v[s, pl.ds(16 * l, 16)] = lax.shift_right_logical(v, 1)
                off_v[s, pl.ds(16 * l, 16)] = (v & 1) * d_model

        def transpose(s, s2):
            @plsc.parallel_loop(0, TB, unroll=8)
            def body(j):
                jv = jnp.broadcast_to(j, (16,)).astype(jnp.int32)
                offs = plsc.load_gather(off_v.at[s], [jv])
                for k in range(d_model // 16):
                    vals = plsc.load_gather(rows_v.at[s], [jv, offs + rvec[k]])
                    plsc.store_scatter(tr_v.at[s2], [rvec[k], jv], vals)

        for t in range(DEPTH):
            idx_cp(t, t).start()
        for t in range(DEPTH):
            idx_cp(t, t).wait()
            compute_p(t)
            gat_cp(t).start()
        idx_cp(DEPTH, 0).start()

        def iter_g(g, s, s2):
            gat_cp(s).wait()

            @pl.when(g >= 2)
            def _():
                out_cp(g - 2, s2).wait()

            transpose(s, s2)
            out_cp(g, s2).start()

            @pl.when(g + DEPTH < per_w)
            def _():
                idx_cp(g + DEPTH, s).wait()
                compute_p(s)
                gat_cp(s).start()

                @pl.when(g + DEPTH + 1 < per_w)
                def _():
                    idx_cp(g + DEPTH + 1, (s + 1) % DEPTH).start()

        def body(m, carry):
            for b in range(DEPTH):
                g = DEPTH * m + b
                iter_g(g, b, b % 2)
            return carry

        lax.fori_loop(0, per_w // DEPTH, body, 0)

        out_cp(per_w - 2, (per_w - 2) % 2).wait()
        out_cp(per_w - 1, (per_w - 1) % 2).wait()

    return embed


def kernel(x, table):
    b, hist = x.shape
    vocab, d_model = table.shape
    xt = x.T.astype(jnp.int32)
    tab2 = table.reshape(vocab // 2, 2 * d_model)
    out_t = _build(hist, b, vocab, d_model)(tab2, xt)
    return jnp.transpose(out_t, (2, 0, 1))


# speedup vs baseline: 1.6701x; 1.6701x over previous
import functools

import jax
import jax.numpy as jnp
from jax import lax
from jax.experimental import pallas as pl
from jax.experimental.pallas import tpu as pltpu
from jax.experimental.pallas import tpu_sc as plsc

NC, NS = 2, 16
NW = NC * NS
TB = 128
PITCH = TB + 1
DEPTH = 4


@functools.lru_cache(maxsize=None)
def _build(hist, batch, vocab, d_model):
    nb = batch // TB
    nblk = hist * nb
    per_w = nblk // NW
    assert nblk % NW == 0 and per_w % DEPTH == 0 and per_w >= 2 * DEPTH

    mesh = plsc.VectorSubcoreMesh(core_axis_name="c", subcore_axis_name="s")

    @functools.partial(
        pl.kernel,
        out_type=jax.ShapeDtypeStruct((hist, d_model, batch), jnp.float32),
        mesh=mesh,
        scratch_types=[
            pltpu.VMEM((DEPTH, TB), jnp.int32),
            pltpu.VMEM((DEPTH, TB), jnp.int32),
            pltpu.VMEM((DEPTH, TB), jnp.int32),
            pltpu.VMEM((DEPTH, TB, 2 * d_model), jnp.float32),
            pltpu.VMEM((2, d_model, PITCH), jnp.float32),
        ] + [pltpu.SemaphoreType.DMA] * (2 * DEPTH + 2),
        compiler_params=pltpu.CompilerParams(
            use_tc_tiling_on_sc=True, needs_layout_passes=False,
            disable_bounds_checks=True),
    )
    def embed(tab2_hbm, xt_hbm, out_hbm, idx_v, off_v, p_v, rows_v, tr_v,
              *sems):
        sem_i = sems[:DEPTH]
        sem_g = sems[DEPTH:2 * DEPTH]
        sem_o = sems[2 * DEPTH:]
        wid = lax.axis_index("s") * NC + lax.axis_index("c")
        iota = lax.iota(jnp.int32, 16)
        rvec = [16 * k + iota for k in range(d_model // 16)]

        def hc(g):
            blk = wid + g * NW
            return blk // nb, blk % nb

        def idx_cp(g, s):
            h, c = hc(g)
            return pltpu.make_async_copy(
                xt_hbm.at[h, pl.ds(c * TB, TB)], idx_v.at[s], sem_i[s])

        def gat_cp(s):
            return pltpu.make_async_copy(
                tab2_hbm.at[p_v.at[s]], rows_v.at[s], sem_g[s])

        def out_cp(g, s2):
            h, c = hc(g)
            return pltpu.make_async_copy(
                tr_v.at[s2, :, pl.ds(0, TB)],
                out_hbm.at[h, :, pl.ds(c * TB, TB)], sem_o[s2])

        def compute_p(s):
            for l in range(TB // 16):
                v = idx_v[s, pl.ds(16 * l, 16)]
                p_v[s, pl.ds(16 * l, 16)] = lax.shift_right_logical(v, 1)
                off_v[s, pl.ds(16 * l, 16)] = (v & 1) * d_model

        def transpose(s, s2):
            @plsc.parallel_loop(0, TB, unroll=8)
            def body(j):
                jv = jnp.broadcast_to(j, (16,)).astype(jnp.int32)
                offs = plsc.load_gather(off_v.at[s], [jv])
                for k in range(d_model // 16):
                    vals = plsc.load_gather(rows_v.at[s], [jv, offs + rvec[k]])
                    plsc.store_scatter(tr_v.at[s2], [rvec[k], jv], vals)

        for t in range(DEPTH):
            idx_cp(t, t).start()
        for t in range(DEPTH):
            idx_cp(t, t).wait()
            compute_p(t)
            gat_cp(t).start()
        idx_cp(DEPTH, 0).start()

        def iter_g(g, s, s2):
            gat_cp(s).wait()

            @pl.when(g >= 2)
            def _():
                out_cp(g - 2, s2).wait()

            out_cp(g, s2).start()

            @pl.when(g + DEPTH < per_w)
            def _():
                idx_cp(g + DEPTH, s).wait()
                compute_p(s)
                gat_cp(s).start()

                @pl.when(g + DEPTH + 1 < per_w)
                def _():
                    idx_cp(g + DEPTH + 1, (s + 1) % DEPTH).start()

        def body(m, carry):
            for b in range(DEPTH):
                g = DEPTH * m + b
                iter_g(g, b, b % 2)
            return carry

        lax.fori_loop(0, per_w // DEPTH, body, 0)

        out_cp(per_w - 2, (per_w - 2) % 2).wait()
        out_cp(per_w - 1, (per_w - 1) % 2).wait()

    return embed


def kernel(x, table):
    b, hist = x.shape
    vocab, d_model = table.shape
    xt = x.T.astype(jnp.int32)
    tab2 = table.reshape(vocab // 2, 2 * d_model)
    out_t = _build(hist, b, vocab, d_model)(tab2, xt)
    return jnp.transpose(out_t, (2, 0, 1))


# speedup vs baseline: 1.8057x; 1.0812x over previous
import functools

import jax
import jax.numpy as jnp
from jax import lax
from jax.experimental import pallas as pl
from jax.experimental.pallas import tpu as pltpu
from jax.experimental.pallas import tpu_sc as plsc

NC, NS = 2, 16
NW = NC * NS
TB = 128
PITCH = TB + 1
DEPTH = 4


@functools.lru_cache(maxsize=None)
def _build(hist, batch, vocab, d_model):
    nb = batch // TB
    nblk = hist * nb
    per_w = nblk // NW
    assert nblk % NW == 0 and per_w % DEPTH == 0 and per_w >= 2 * DEPTH
    dsub = d_model // 8

    mesh = plsc.VectorSubcoreMesh(core_axis_name="c", subcore_axis_name="s")

    @functools.partial(
        pl.kernel,
        out_type=jax.ShapeDtypeStruct((hist, dsub, nb, 8, TB), jnp.float32),
        mesh=mesh,
        scratch_types=[
            pltpu.VMEM((DEPTH, TB), jnp.int32),
            pltpu.VMEM((DEPTH, TB, 2 * d_model), jnp.float32),
            pltpu.VMEM((2, dsub, 8, PITCH), jnp.float32),
        ] + [pltpu.SemaphoreType.DMA] * (2 * DEPTH + 2),
        compiler_params=pltpu.CompilerParams(
            use_tc_tiling_on_sc=False, needs_layout_passes=False,
            disable_bounds_checks=True),
    )
    def embed(tabp_hbm, xt_hbm, out_hbm, idx_v, rows_v, tr_v, *sems):
        sem_i = sems[:DEPTH]
        sem_g = sems[DEPTH:2 * DEPTH]
        sem_o = sems[2 * DEPTH:]
        wid = lax.axis_index("s") * NC + lax.axis_index("c")
        iota = lax.iota(jnp.int32, 16)
        rvec = [16 * k + iota for k in range(d_model // 16)]
        rhi = [lax.shift_right_logical(r, 3) for r in rvec]
        rlo = [r & 7 for r in rvec]

        def hc(g):
            blk = wid + g * NW
            return blk // nb, blk % nb

        def idx_cp(g, s):
            h, c = hc(g)
            return pltpu.make_async_copy(
                xt_hbm.at[h, pl.ds(c * TB, TB)], idx_v.at[s], sem_i[s])

        def gat_cp(s):
            return pltpu.make_async_copy(
                tabp_hbm.at[idx_v.at[s]], rows_v.at[s], sem_g[s])

        def out_cp(g, s2):
            h, c = hc(g)
            return pltpu.make_async_copy(
                tr_v.at[s2, :, :, pl.ds(0, TB)],
                out_hbm.at[h, :, c], sem_o[s2])

        def transpose(s, s2):
            @plsc.parallel_loop(0, TB, unroll=8)
            def body(j):
                jv = jnp.broadcast_to(j, (16,)).astype(jnp.int32)
                for k in range(d_model // 16):
                    vals = plsc.load_gather(rows_v.at[s], [jv, rvec[k]])
                    plsc.store_scatter(tr_v.at[s2], [rhi[k], rlo[k], jv], vals)

        for t in range(DEPTH):
            idx_cp(t, t).start()
        for t in range(DEPTH):
            idx_cp(t, t).wait()
            gat_cp(t).start()
        idx_cp(DEPTH, 0).start()

        def iter_g(g, s, s2):
            gat_cp(s).wait()

            @pl.when(g >= 2)
            def _():
                out_cp(g - 2, s2).wait()

            transpose(s, s2)
            out_cp(g, s2).start()

            @pl.when(g + DEPTH < per_w)
            def _():
                idx_cp(g + DEPTH, s).wait()
                gat_cp(s).start()

                @pl.when(g + DEPTH + 1 < per_w)
                def _():
                    idx_cp(g + DEPTH + 1, (s + 1) % DEPTH).start()

        def body(m, carry):
            for b in range(DEPTH):
                iter_g(DEPTH * m + b, b, b % 2)
            return carry

        lax.fori_loop(0, per_w // DEPTH, body, 0)

        out_cp(per_w - 2, (per_w - 2) % 2).wait()
        out_cp(per_w - 1, (per_w - 1) % 2).wait()

    return embed


def kernel(x, table):
    b, hist = x.shape
    vocab, d_model = table.shape
    xt = x.T.astype(jnp.int32)
    tabp = jnp.pad(table, ((0, 0), (0, 128 - d_model)))
    out5 = _build(hist, b, vocab, d_model)(tabp, xt)
    return out5.transpose(2, 4, 0, 1, 3).reshape(b, hist, d_model)


# speedup vs baseline: 1.8063x; 1.0003x over previous
import functools

import jax
import jax.numpy as jnp
from jax import lax
from jax.experimental import pallas as pl
from jax.experimental.pallas import tpu as pltpu
from jax.experimental.pallas import tpu_sc as plsc

NC, NS = 2, 16
NW = NC * NS
TB = 128
PITCH = TB + 1
DEPTH = 4


@functools.lru_cache(maxsize=None)
def _build(hist, batch, vocab, d_model):
    nb = batch // TB
    nblk = hist * nb
    per_w = nblk // NW
    assert nblk % NW == 0 and per_w % DEPTH == 0 and per_w >= 2 * DEPTH
    dsub = d_model // 8

    mesh = plsc.VectorSubcoreMesh(core_axis_name="c", subcore_axis_name="s")

    @functools.partial(
        pl.kernel,
        out_type=jax.ShapeDtypeStruct((hist, dsub, nb, 8, TB), jnp.float32),
        mesh=mesh,
        scratch_types=[
            pltpu.VMEM((DEPTH, TB), jnp.int32),
            pltpu.VMEM((DEPTH, TB, 2 * d_model), jnp.float32),
            pltpu.VMEM((2, dsub, 8, PITCH), jnp.float32),
        ] + [pltpu.SemaphoreType.DMA] * (2 * DEPTH + 2),
        compiler_params=pltpu.CompilerParams(
            use_tc_tiling_on_sc=False, needs_layout_passes=False,
            disable_bounds_checks=True),
    )
    def embed(tabp_hbm, xt_hbm, out_hbm, idx_v, rows_v, tr_v, *sems):
        sem_i = sems[:DEPTH]
        sem_g = sems[DEPTH:2 * DEPTH]
        sem_o = sems[2 * DEPTH:]
        wid = lax.axis_index("s") * NC + lax.axis_index("c")
        iota = lax.iota(jnp.int32, 16)
        rvec = [16 * k + iota for k in range(d_model // 16)]
        rhi = [lax.shift_right_logical(r, 3) for r in rvec]
        rlo = [r & 7 for r in rvec]

        def hc(g):
            blk = wid + g * NW
            return blk // nb, blk % nb

        def idx_cp(g, s):
            h, c = hc(g)
            return pltpu.make_async_copy(
                xt_hbm.at[h, pl.ds(c * TB, TB)], idx_v.at[s], sem_i[s])

        def gat_cp(s):
            return pltpu.make_async_copy(
                tabp_hbm.at[idx_v.at[s]], rows_v.at[s], sem_g[s])

        def out_cp(g, s2):
            h, c = hc(g)
            return pltpu.make_async_copy(
                tr_v.at[s2, :, :, pl.ds(0, TB)],
                out_hbm.at[h, :, c], sem_o[s2])

        def transpose(s, s2):
            @plsc.parallel_loop(0, TB, unroll=16)
            def body(j):
                jv = jnp.broadcast_to(j, (16,)).astype(jnp.int32)
                for k in range(d_model // 16):
                    vals = plsc.load_gather(rows_v.at[s], [jv, rvec[k]])
                    plsc.store_scatter(tr_v.at[s2], [rhi[k], rlo[k], jv], vals)

        for t in range(DEPTH):
            idx_cp(t, t).start()
        for t in range(DEPTH):
            idx_cp(t, t).wait()
            gat_cp(t).start()
        idx_cp(DEPTH, 0).start()

        def iter_g(g, s, s2):
            gat_cp(s).wait()

            @pl.when(g >= 2)
            def _():
                out_cp(g - 2, s2).wait()

            transpose(s, s2)
            out_cp(g, s2).start()

            @pl.when(g + DEPTH < per_w)
            def _():
                idx_cp(g + DEPTH, s).wait()
                gat_cp(s).start()

                @pl.when(g + DEPTH + 1 < per_w)
                def _():
                    idx_cp(g + DEPTH + 1, (s + 1) % DEPTH).start()

        def body(m, carry):
            for b in range(DEPTH):
                iter_g(DEPTH * m + b, b, b % 2)
            return carry

        lax.fori_loop(0, per_w // DEPTH, body, 0)

        out_cp(per_w - 2, (per_w - 2) % 2).wait()
        out_cp(per_w - 1, (per_w - 1) % 2).wait()

    return embed


def kernel(x, table):
    b, hist = x.shape
    vocab, d_model = table.shape
    xt = x.T.astype(jnp.int32)
    tabp = jnp.pad(table, ((0, 0), (0, 128 - d_model)))
    out5 = _build(hist, b, vocab, d_model)(tabp, xt)
    return out5.transpose(2, 4, 0, 1, 3).reshape(b, hist, d_model)
